# baseline (speedup 1.0000x reference)
import functools

import jax
import jax.numpy as jnp
from jax import lax
from jax.experimental import pallas as pl
from jax.experimental.pallas import tpu as pltpu
from jax.experimental.pallas import tpu_sc as plsc

N_OUT = 400000
D = 128
NC = 2
NS = 16
NW = NC * NS
BLK = 128
NBLK = N_OUT // BLK
BLK_PER_W = -(-NBLK // NW)

_mesh = plsc.VectorSubcoreMesh(core_axis_name="c", subcore_axis_name="s")

NBUF = 6
K = NBUF - 1


@functools.partial(
    pl.kernel,
    mesh=_mesh,
    out_type=jax.ShapeDtypeStruct((N_OUT, D), jnp.float32),
    scratch_types=[
        pltpu.VMEM((BLK_PER_W * BLK,), jnp.int32),
        pltpu.VMEM((NBUF, BLK, D), jnp.float32),
        pltpu.SemaphoreType.DMA((NBUF,)),
        pltpu.SemaphoreType.DMA((NBUF,)),
    ],
)
def _gather_kernel(table_hbm, idx_hbm, out_hbm, idx_v, rows_v, gsem, wsem):
    wid = lax.axis_index("s") * NC + lax.axis_index("c")
    blk0 = wid * BLK_PER_W
    n_w = jnp.maximum(jnp.minimum(NBLK - blk0, BLK_PER_W), 0)
    start_blk = jnp.minimum(blk0, NBLK - BLK_PER_W)
    row_off = blk0 - start_blk
    pltpu.sync_copy(idx_hbm.at[pl.ds(start_blk * BLK, BLK_PER_W * BLK)], idx_v)

    def idx_blk(j):
        return idx_v.at[pl.ds((j + row_off) * BLK, BLK)]

    def gather_start(j, slot):
        pltpu.async_copy(
            table_hbm.at[idx_blk(j)], rows_v.at[slot], gsem.at[slot])

    def gather_wait(j, slot):
        pltpu.make_async_copy(
            table_hbm.at[idx_blk(j)], rows_v.at[slot], gsem.at[slot]).wait()

    def write_start(j, slot):
        pltpu.async_copy(
            rows_v.at[slot], out_hbm.at[pl.ds((blk0 + j) * BLK, BLK)],
            wsem.at[slot])

    def write_wait(slot):
        pltpu.make_async_copy(
            rows_v.at[slot], out_hbm.at[pl.ds(0, BLK)], wsem.at[slot]).wait()

    for jj in range(K):
        @pl.when(jj < n_w)
        def _(jj=jj):
            gather_start(jj, jj)

    def body(j, carry):
        b = lax.rem(j, NBUF)
        gather_wait(j, b)
        write_start(j, b)

        @pl.when(j + K < n_w)
        def _():
            bn = lax.rem(j + K, NBUF)

            @pl.when(j >= 1)
            def _():
                write_wait(bn)

            gather_start(j + K, bn)

        return carry

    lax.fori_loop(0, n_w, body, 0)

    for b in range(NBUF):
        @pl.when(b < n_w)
        def _(b=b):
            write_wait(b)


def kernel(inputs, vt_replace, vt_map):
    del vt_replace
    return _gather_kernel(inputs, vt_map)


# speedup vs baseline: 1.0039x; 1.0039x over previous
import functools

import jax
import jax.numpy as jnp
from jax import lax
from jax.experimental import pallas as pl
from jax.experimental.pallas import tpu as pltpu
from jax.experimental.pallas import tpu_sc as plsc

N_OUT = 400000
D = 128
NC = 2
NS = 16
NW = NC * NS
BLK = 128
NBLK = N_OUT // BLK
BLK_PER_W = -(-NBLK // NW)

_mesh = plsc.VectorSubcoreMesh(core_axis_name="c", subcore_axis_name="s")

NBUF = 7
K = NBUF - 2


@functools.partial(
    pl.kernel,
    mesh=_mesh,
    out_type=jax.ShapeDtypeStruct((N_OUT, D), jnp.float32),
    scratch_types=[
        pltpu.VMEM((BLK_PER_W * BLK,), jnp.int32),
        pltpu.VMEM((NBUF, BLK, D), jnp.float32),
        pltpu.SemaphoreType.DMA((NBUF,)),
        pltpu.SemaphoreType.DMA((NBUF,)),
    ],
)
def _gather_kernel(table_hbm, idx_hbm, out_hbm, idx_v, rows_v, gsem, wsem):
    wid = lax.axis_index("s") * NC + lax.axis_index("c")
    blk0 = wid * BLK_PER_W
    n_w = jnp.maximum(jnp.minimum(NBLK - blk0, BLK_PER_W), 0)
    start_blk = jnp.minimum(blk0, NBLK - BLK_PER_W)
    row_off = blk0 - start_blk
    pltpu.sync_copy(idx_hbm.at[pl.ds(start_blk * BLK, BLK_PER_W * BLK)], idx_v)

    def idx_blk(j):
        return idx_v.at[pl.ds((j + row_off) * BLK, BLK)]

    def gather_start(j, slot):
        pltpu.async_copy(
            table_hbm.at[idx_blk(j)], rows_v.at[slot], gsem.at[slot])

    def gather_wait(j, slot):
        pltpu.make_async_copy(
            table_hbm.at[idx_blk(j)], rows_v.at[slot], gsem.at[slot]).wait()

    def write_start(j, slot):
        pltpu.async_copy(
            rows_v.at[slot], out_hbm.at[pl.ds((blk0 + j) * BLK, BLK)],
            wsem.at[slot])

    def write_wait(slot):
        pltpu.make_async_copy(
            rows_v.at[slot], out_hbm.at[pl.ds(0, BLK)], wsem.at[slot]).wait()

    for jj in range(K):
        @pl.when(jj < n_w)
        def _(jj=jj):
            gather_start(jj, jj)

    def body(j, carry):
        b = lax.rem(j, NBUF)
        gather_wait(j, b)
        write_start(j, b)

        @pl.when(j + K < n_w)
        def _():
            bn = lax.rem(j + K, NBUF)

            @pl.when(j + K - NBUF >= 0)
            def _():
                write_wait(bn)

            gather_start(j + K, bn)

        return carry

    lax.fori_loop(0, n_w, body, 0)

    for b in range(NBUF):
        @pl.when(b < n_w)
        def _(b=b):
            write_wait(b)


def kernel(inputs, vt_replace, vt_map):
    del vt_replace
    return _gather_kernel(inputs, vt_map)
